# baseline (speedup 1.0000x reference)
import functools

import jax
import jax.numpy as jnp
from jax import lax
from jax.experimental import pallas as pl
from jax.experimental.pallas import tpu as pltpu
from jax.experimental.pallas import tpu_sc as plsc

N = 100000
TOP_N = 10
NEG = -1e30

BLK_G = 8192
BLK_V = 4096

L = 16
NTILE = 16


def _sims_body(x_ref, q_ref, o_ref, *, d, blk):
    x = x_ref[...]
    q = q_ref[...]
    dot = jnp.sum(x * q, axis=1)
    sq = jnp.sum(x * x, axis=1)
    qn = jnp.maximum(jnp.sqrt(jnp.sum(q * q)), 1e-8)
    xn = jnp.maximum(jnp.sqrt(sq), 1e-8)
    sims = dot / (xn * qn)
    i = pl.program_id(0)
    row = i * blk + lax.broadcasted_iota(jnp.int32, (1, blk), 1)
    o_ref[...] = jnp.where(row < N, sims.reshape(1, blk), NEG).reshape(1, 1, blk)


def _sims(x, q, blk):
    d = x.shape[1]
    nblk = (N + blk - 1) // blk
    return pl.pallas_call(
        functools.partial(_sims_body, d=d, blk=blk),
        grid=(nblk,),
        in_specs=[
            pl.BlockSpec((blk, d), lambda i: (i, 0)),
            pl.BlockSpec((1, d), lambda i: (0, 0)),
        ],
        out_specs=pl.BlockSpec((1, 1, blk), lambda i: (i, 0, 0)),
        out_shape=jax.ShapeDtypeStruct((nblk, 1, blk), jnp.float32),
    )(x, q).reshape(nblk * blk)


def _batcher_pairs(n):
    pairs = []
    p = 1
    while p < n:
        k = p
        while k >= 1:
            for j in range(k % p, n - k, 2 * k):
                for i in range(0, min(k, n - j - k)):
                    if (i + j) // (2 * p) == (i + j + k) // (2 * p):
                        pairs.append((i + j, i + j + k))
            k //= 2
        p *= 2
    return pairs


_PAIRS16 = _batcher_pairs(16)

_GDN = lax.GatherDimensionNumbers(offset_dims=(), collapsed_slice_dims=(0,),
                                  start_index_map=(0,))


def _perm(x, idx):
    return lax.gather(x, idx.reshape(L, 1), _GDN, (1,),
                      mode=lax.GatherScatterMode.PROMISE_IN_BOUNDS)


def _ce(av, ai, bv, bi):
    take = bv > av
    return (jnp.where(take, bv, av), jnp.where(take, bi, ai),
            jnp.where(take, av, bv), jnp.where(take, ai, bi))


def _lane_merge(lane, av, ai, bv, bi):
    brv = lax.rev(bv, (0,))
    bri = lax.rev(bi, (0,))
    take = brv > av
    m = jnp.where(take, brv, av)
    mi = jnp.where(take, bri, ai)
    for d in (8, 4, 2, 1):
        p = lane ^ d
        pv = _perm(m, p)
        pi = _perm(mi, p)
        low = (lane & d) == 0
        take = (m > pv) | ((m == pv) & (mi < pi))
        mxv = jnp.where(take, m, pv)
        mxi = jnp.where(take, mi, pi)
        mnv = jnp.where(take, pv, m)
        mni = jnp.where(take, pi, mi)
        m = jnp.where(low, mxv, mnv)
        mi = jnp.where(low, mxi, mni)
    return m, mi


def _sc_topk_factory(npad):
    chunk = npad // NTILE
    ngroup = chunk // (16 * L)
    assert ngroup * 16 * L * NTILE == npad
    mesh = plsc.VectorSubcoreMesh(core_axis_name="c", subcore_axis_name="s")
    f32 = jnp.float32
    i32 = jnp.int32
    out_type = [jax.ShapeDtypeStruct((3 * L,), f32),
                jax.ShapeDtypeStruct((3 * L,), i32)]
    scratch = [
        pltpu.VMEM((chunk,), f32),
        pltpu.VMEM((16 * L,), f32),
        pltpu.VMEM((16 * L,), i32),
        pltpu.VMEM((L,), f32),
        pltpu.VMEM((L,), i32),
        pltpu.VMEM((L,), f32),
        pltpu.VMEM_SHARED((3 * NTILE * L,), f32),
        pltpu.VMEM_SHARED((3 * NTILE * L,), i32),
        pltpu.VMEM_SHARED((3 * NTILE * L,), f32),
        pltpu.VMEM((NTILE * L,), f32),
        pltpu.VMEM((NTILE * L,), i32),
        pltpu.VMEM((NTILE * L,), f32),
        pltpu.VMEM((L,), f32),
        pltpu.VMEM((L,), i32),
    ]

    @functools.partial(pl.kernel, mesh=mesh, out_type=out_type,
                       scratch_types=scratch)
    def sc_topk(src, orv, ori,
                c_buf, accv, acci, tv, ti, te, sh_v, sh_i, sh_e,
                g_v, g_i, g_e, ow, oi):
        cid = lax.axis_index("c")
        sid = lax.axis_index("s")
        base = sid * chunk
        lane = lax.broadcasted_iota(i32, (L,), 0)
        neg = jnp.full((L,), NEG, f32)
        zero_i = jnp.zeros((L,), i32)

        if True:
            def branch_body(b, _):
                pltpu.sync_copy(src.at[pl.ds(b * npad + base, chunk)], c_buf)
                for j in range(16):
                    accv[pl.ds(j * L, L)] = neg
                    acci[pl.ds(j * L, L)] = zero_i

                def group_body(g, esum):
                    off = g * (16 * L)
                    vsl, isl = [], []
                    for jj in range(16):
                        v = c_buf[pl.ds(off + jj * L, L)]
                        esum = esum + jnp.exp(v)
                        vsl.append(v)
                        isl.append(base + off + jj * L + lane)
                    for (a, b2) in _PAIRS16:
                        vsl[a], isl[a], vsl[b2], isl[b2] = _ce(
                            vsl[a], isl[a], vsl[b2], isl[b2])
                    M, Mi = [], []
                    for j in range(16):
                        A = accv[pl.ds(j * L, L)]
                        Ai = acci[pl.ds(j * L, L)]
                        take = vsl[15 - j] > A
                        M.append(jnp.where(take, vsl[15 - j], A))
                        Mi.append(jnp.where(take, isl[15 - j], Ai))
                    for d in (8, 4, 2, 1):
                        for j in range(16):
                            p = j ^ d
                            if j < p:
                                M[j], Mi[j], M[p], Mi[p] = _ce(
                                    M[j], Mi[j], M[p], Mi[p])
                    for j in range(16):
                        accv[pl.ds(j * L, L)] = M[j]
                        acci[pl.ds(j * L, L)] = Mi[j]
                    return esum

                esum = lax.fori_loop(0, ngroup, group_body,
                                     jnp.zeros((L,), f32))

                A = [accv[pl.ds(j * L, L)] for j in range(16)]
                Ai = [acci[pl.ds(j * L, L)] for j in range(16)]
                out_v = neg
                out_i = zero_i
                for k in range(16):
                    mv, mi = A[0], Ai[0]
                    for d in (1, 2, 4, 8):
                        p = lane ^ d
                        pv = _perm(mv, p)
                        pi = _perm(mi, p)
                        take = (mv > pv) | ((mv == pv) & (mi < pi))
                        mv = jnp.where(take, mv, pv)
                        mi = jnp.where(take, mi, pi)
                    out_v = jnp.where(lane == k, mv, out_v)
                    out_i = jnp.where(lane == k, mi, out_i)
                    pop = (A[0] == mv) & (Ai[0] == mi)
                    for j in range(15):
                        A[j] = jnp.where(pop, A[j + 1], A[j])
                        Ai[j] = jnp.where(pop, Ai[j + 1], Ai[j])
                    A[15] = jnp.where(pop, neg, A[15])
                    Ai[15] = jnp.where(pop, zero_i, Ai[15])

                tv[...] = out_v
                ti[...] = out_i
                te[...] = esum
                soff = b * (NTILE * L) + sid * L
                pltpu.sync_copy(tv, sh_v.at[pl.ds(soff, L)])
                pltpu.sync_copy(ti, sh_i.at[pl.ds(soff, L)])
                pltpu.sync_copy(te, sh_e.at[pl.ds(soff, L)])
                return _

            lax.fori_loop(0, 3, branch_body, jnp.int32(0))

        plsc.subcore_barrier()

        @pl.when(jnp.logical_and(cid == 0, sid == 0))
        def _():
            if True:
                def out_body(b, _):
                    soff = b * (NTILE * L)
                    pltpu.sync_copy(sh_v.at[pl.ds(soff, NTILE * L)], g_v)
                    pltpu.sync_copy(sh_i.at[pl.ds(soff, NTILE * L)], g_i)
                    pltpu.sync_copy(sh_e.at[pl.ds(soff, NTILE * L)], g_e)
                    esum = g_e[pl.ds(0, L)]
                    for t in range(1, NTILE):
                        esum = esum + g_e[pl.ds(t * L, L)]
                    for d in (1, 2, 4, 8):
                        esum = esum + _perm(esum, lane ^ d)
                    cv = [g_v[pl.ds(t * L, L)] for t in range(NTILE)]
                    ci = [g_i[pl.ds(t * L, L)] for t in range(NTILE)]
                    while len(cv) > 1:
                        nv, ni = [], []
                        for t in range(0, len(cv), 2):
                            m, mi = _lane_merge(lane, cv[t], ci[t],
                                                cv[t + 1], ci[t + 1])
                            nv.append(m)
                            ni.append(mi)
                        cv, ci = nv, ni
                    ow[...] = jnp.exp(cv[0]) / esum
                    oi[...] = ci[0]
                    pltpu.sync_copy(ow, orv.at[pl.ds(b * L, L)])
                    pltpu.sync_copy(oi, ori.at[pl.ds(b * L, L)])
                    return _

                lax.fori_loop(0, 3, out_body, jnp.int32(0))

    return sc_topk


_NPAD_G = ((N + BLK_G - 1) // BLK_G) * BLK_G
_NPAD_V = ((N + BLK_V - 1) // BLK_V) * BLK_V
_sc_topk_g = _sc_topk_factory(_NPAD_G)
_sc_topk_v = _sc_topk_factory(_NPAD_V)


def kernel(gemini_sections, gemini_chapters, gemini_pages,
           voyager_sections, voyager_chapters, voyager_pages,
           gemini_query_embedding, voyager_query_embedding):
    gq = gemini_query_embedding
    vq = voyager_query_embedding
    gsims = jnp.stack([_sims(gemini_sections, gq, BLK_G),
                       _sims(gemini_chapters, gq, BLK_G),
                       _sims(gemini_pages, gq, BLK_G)])
    vsims = jnp.stack([_sims(voyager_sections, vq, BLK_V),
                       _sims(voyager_chapters, vq, BLK_V),
                       _sims(voyager_pages, vq, BLK_V)])
    gv, gi = _sc_topk_g(gsims.reshape(-1))
    vv, vi = _sc_topk_v(vsims.reshape(-1))
    gv = gv.reshape(3, L)
    gi = gi.reshape(3, L)
    vv = vv.reshape(3, L)
    vi = vi.reshape(3, L)
    return (gv[0, :TOP_N], gi[0, :TOP_N], gv[1, :TOP_N], gi[1, :TOP_N],
            gv[2, :TOP_N], gi[2, :TOP_N], vv[0, :TOP_N], vi[0, :TOP_N],
            vv[1, :TOP_N], vi[1, :TOP_N], vv[2, :TOP_N], vi[2, :TOP_N])


# speedup vs baseline: 1.0158x; 1.0158x over previous
import functools

import jax
import jax.numpy as jnp
from jax import lax
from jax.experimental import pallas as pl
from jax.experimental.pallas import tpu as pltpu
from jax.experimental.pallas import tpu_sc as plsc

N = 100000
TOP_N = 10
NEG = -1e30

BLK_G = 4096
BLK_V = 4096

L = 16
NTILE = 16


def _sims_body(x_ref, q_ref, o_ref, *, d, blk):
    x = x_ref[...]
    q = q_ref[...]
    dot = jnp.sum(x * q, axis=1)
    sq = jnp.sum(x * x, axis=1)
    qn = jnp.maximum(jnp.sqrt(jnp.sum(q * q)), 1e-8)
    xn = jnp.maximum(jnp.sqrt(sq), 1e-8)
    sims = dot / (xn * qn)
    i = pl.program_id(0)
    row = i * blk + lax.broadcasted_iota(jnp.int32, (1, blk), 1)
    o_ref[...] = jnp.where(row < N, sims.reshape(1, blk), NEG).reshape(1, 1, blk)


def _sims(x, q, blk):
    d = x.shape[1]
    nblk = (N + blk - 1) // blk
    return pl.pallas_call(
        functools.partial(_sims_body, d=d, blk=blk),
        grid=(nblk,),
        in_specs=[
            pl.BlockSpec((blk, d), lambda i: (i, 0)),
            pl.BlockSpec((1, d), lambda i: (0, 0)),
        ],
        out_specs=pl.BlockSpec((1, 1, blk), lambda i: (i, 0, 0)),
        out_shape=jax.ShapeDtypeStruct((nblk, 1, blk), jnp.float32),
    )(x, q).reshape(nblk * blk)


def _batcher_pairs(n):
    pairs = []
    p = 1
    while p < n:
        k = p
        while k >= 1:
            for j in range(k % p, n - k, 2 * k):
                for i in range(0, min(k, n - j - k)):
                    if (i + j) // (2 * p) == (i + j + k) // (2 * p):
                        pairs.append((i + j, i + j + k))
            k //= 2
        p *= 2
    return pairs


_PAIRS16 = _batcher_pairs(16)

_GDN = lax.GatherDimensionNumbers(offset_dims=(), collapsed_slice_dims=(0,),
                                  start_index_map=(0,))


def _perm(x, idx):
    return lax.gather(x, idx.reshape(L, 1), _GDN, (1,),
                      mode=lax.GatherScatterMode.PROMISE_IN_BOUNDS)


def _ce(av, ai, bv, bi):
    take = bv > av
    return (jnp.where(take, bv, av), jnp.where(take, bi, ai),
            jnp.where(take, av, bv), jnp.where(take, ai, bi))


def _lane_merge(lane, av, ai, bv, bi):
    brv = lax.rev(bv, (0,))
    bri = lax.rev(bi, (0,))
    take = brv > av
    m = jnp.where(take, brv, av)
    mi = jnp.where(take, bri, ai)
    for d in (8, 4, 2, 1):
        p = lane ^ d
        pv = _perm(m, p)
        pi = _perm(mi, p)
        low = (lane & d) == 0
        take = (m > pv) | ((m == pv) & (mi < pi))
        mxv = jnp.where(take, m, pv)
        mxi = jnp.where(take, mi, pi)
        mnv = jnp.where(take, pv, m)
        mni = jnp.where(take, pi, mi)
        m = jnp.where(low, mxv, mnv)
        mi = jnp.where(low, mxi, mni)
    return m, mi


def _sc_topk_factory(npad):
    chunk = npad // NTILE
    ngroup = chunk // (16 * L)
    assert ngroup * 16 * L * NTILE == npad
    mesh = plsc.VectorSubcoreMesh(core_axis_name="c", subcore_axis_name="s")
    f32 = jnp.float32
    i32 = jnp.int32
    out_type = [jax.ShapeDtypeStruct((3 * L,), f32),
                jax.ShapeDtypeStruct((3 * L,), i32)]
    scratch = [
        pltpu.VMEM((chunk,), f32),
        pltpu.VMEM((16 * L,), f32),
        pltpu.VMEM((16 * L,), i32),
        pltpu.VMEM((L,), f32),
        pltpu.VMEM((L,), i32),
        pltpu.VMEM((L,), f32),
        pltpu.VMEM_SHARED((3 * NTILE * L,), f32),
        pltpu.VMEM_SHARED((3 * NTILE * L,), i32),
        pltpu.VMEM_SHARED((3 * NTILE * L,), f32),
        pltpu.VMEM((NTILE * L,), f32),
        pltpu.VMEM((NTILE * L,), i32),
        pltpu.VMEM((NTILE * L,), f32),
        pltpu.VMEM((L,), f32),
        pltpu.VMEM((L,), i32),
    ]

    @functools.partial(pl.kernel, mesh=mesh, out_type=out_type,
                       scratch_types=scratch)
    def sc_topk(src, orv, ori,
                c_buf, accv, acci, tv, ti, te, sh_v, sh_i, sh_e,
                g_v, g_i, g_e, ow, oi):
        cid = lax.axis_index("c")
        sid = lax.axis_index("s")
        base = sid * chunk
        lane = lax.broadcasted_iota(i32, (L,), 0)
        neg = jnp.full((L,), NEG, f32)
        zero_i = jnp.zeros((L,), i32)

        if True:
            def branch_body(b, _):
                pltpu.sync_copy(src.at[pl.ds(b * npad + base, chunk)], c_buf)
                for j in range(16):
                    accv[pl.ds(j * L, L)] = neg
                    acci[pl.ds(j * L, L)] = zero_i

                def group_body(g, esum):
                    off = g * (16 * L)
                    vsl, isl = [], []
                    for jj in range(16):
                        v = c_buf[pl.ds(off + jj * L, L)]
                        esum = esum + jnp.exp(v)
                        vsl.append(v)
                        isl.append(base + off + jj * L + lane)
                    for (a, b2) in _PAIRS16:
                        vsl[a], isl[a], vsl[b2], isl[b2] = _ce(
                            vsl[a], isl[a], vsl[b2], isl[b2])
                    M, Mi = [], []
                    for j in range(16):
                        A = accv[pl.ds(j * L, L)]
                        Ai = acci[pl.ds(j * L, L)]
                        take = vsl[15 - j] > A
                        M.append(jnp.where(take, vsl[15 - j], A))
                        Mi.append(jnp.where(take, isl[15 - j], Ai))
                    for d in (8, 4, 2, 1):
                        for j in range(16):
                            p = j ^ d
                            if j < p:
                                M[j], Mi[j], M[p], Mi[p] = _ce(
                                    M[j], Mi[j], M[p], Mi[p])
                    for j in range(16):
                        accv[pl.ds(j * L, L)] = M[j]
                        acci[pl.ds(j * L, L)] = Mi[j]
                    return esum

                esum = lax.fori_loop(0, ngroup, group_body,
                                     jnp.zeros((L,), f32))

                A = [accv[pl.ds(j * L, L)] for j in range(16)]
                Ai = [acci[pl.ds(j * L, L)] for j in range(16)]
                out_v = neg
                out_i = zero_i
                for k in range(16):
                    mv, mi = A[0], Ai[0]
                    for d in (1, 2, 4, 8):
                        p = lane ^ d
                        pv = _perm(mv, p)
                        pi = _perm(mi, p)
                        take = (mv > pv) | ((mv == pv) & (mi < pi))
                        mv = jnp.where(take, mv, pv)
                        mi = jnp.where(take, mi, pi)
                    out_v = jnp.where(lane == k, mv, out_v)
                    out_i = jnp.where(lane == k, mi, out_i)
                    pop = (A[0] == mv) & (Ai[0] == mi)
                    for j in range(15):
                        A[j] = jnp.where(pop, A[j + 1], A[j])
                        Ai[j] = jnp.where(pop, Ai[j + 1], Ai[j])
                    A[15] = jnp.where(pop, neg, A[15])
                    Ai[15] = jnp.where(pop, zero_i, Ai[15])

                tv[...] = out_v
                ti[...] = out_i
                te[...] = esum
                soff = b * (NTILE * L) + sid * L
                pltpu.sync_copy(tv, sh_v.at[pl.ds(soff, L)])
                pltpu.sync_copy(ti, sh_i.at[pl.ds(soff, L)])
                pltpu.sync_copy(te, sh_e.at[pl.ds(soff, L)])
                return _

            lax.fori_loop(0, 3, branch_body, jnp.int32(0))

        plsc.subcore_barrier()

        @pl.when(jnp.logical_and(cid == 0, sid == 0))
        def _():
            if True:
                def out_body(b, _):
                    soff = b * (NTILE * L)
                    pltpu.sync_copy(sh_v.at[pl.ds(soff, NTILE * L)], g_v)
                    pltpu.sync_copy(sh_i.at[pl.ds(soff, NTILE * L)], g_i)
                    pltpu.sync_copy(sh_e.at[pl.ds(soff, NTILE * L)], g_e)
                    esum = g_e[pl.ds(0, L)]
                    for t in range(1, NTILE):
                        esum = esum + g_e[pl.ds(t * L, L)]
                    for d in (1, 2, 4, 8):
                        esum = esum + _perm(esum, lane ^ d)
                    cv = [g_v[pl.ds(t * L, L)] for t in range(NTILE)]
                    ci = [g_i[pl.ds(t * L, L)] for t in range(NTILE)]
                    while len(cv) > 1:
                        nv, ni = [], []
                        for t in range(0, len(cv), 2):
                            m, mi = _lane_merge(lane, cv[t], ci[t],
                                                cv[t + 1], ci[t + 1])
                            nv.append(m)
                            ni.append(mi)
                        cv, ci = nv, ni
                    ow[...] = jnp.exp(cv[0]) / esum
                    oi[...] = ci[0]
                    pltpu.sync_copy(ow, orv.at[pl.ds(b * L, L)])
                    pltpu.sync_copy(oi, ori.at[pl.ds(b * L, L)])
                    return _

                lax.fori_loop(0, 3, out_body, jnp.int32(0))

    return sc_topk


_NPAD_G = ((N + BLK_G - 1) // BLK_G) * BLK_G
_NPAD_V = ((N + BLK_V - 1) // BLK_V) * BLK_V
_sc_topk_g = _sc_topk_factory(_NPAD_G)
_sc_topk_v = _sc_topk_factory(_NPAD_V)


def kernel(gemini_sections, gemini_chapters, gemini_pages,
           voyager_sections, voyager_chapters, voyager_pages,
           gemini_query_embedding, voyager_query_embedding):
    gq = gemini_query_embedding
    vq = voyager_query_embedding
    gsims = jnp.stack([_sims(gemini_sections, gq, BLK_G),
                       _sims(gemini_chapters, gq, BLK_G),
                       _sims(gemini_pages, gq, BLK_G)])
    vsims = jnp.stack([_sims(voyager_sections, vq, BLK_V),
                       _sims(voyager_chapters, vq, BLK_V),
                       _sims(voyager_pages, vq, BLK_V)])
    gv, gi = _sc_topk_g(gsims.reshape(-1))
    vv, vi = _sc_topk_v(vsims.reshape(-1))
    gv = gv.reshape(3, L)
    gi = gi.reshape(3, L)
    vv = vv.reshape(3, L)
    vi = vi.reshape(3, L)
    return (gv[0, :TOP_N], gi[0, :TOP_N], gv[1, :TOP_N], gi[1, :TOP_N],
            gv[2, :TOP_N], gi[2, :TOP_N], vv[0, :TOP_N], vi[0, :TOP_N],
            vv[1, :TOP_N], vi[1, :TOP_N], vv[2, :TOP_N], vi[2, :TOP_N])
